# baseline (speedup 1.0000x reference)
import functools

import jax
import jax.numpy as jnp
from jax import lax
from jax.experimental import pallas as pl
from jax.experimental.pallas import tpu as pltpu
from jax.experimental.pallas import tpu_sc as plsc

EMB = 64
ROWW = 128
BATCH = 16384
NC = 2
NS = 16
NW = NC * NS
BPW = BATCH // NW
CHUNK = 128
NCHUNK = BPW // CHUNK
NGROUP = BPW // 16
LANES = 16
PSTRIDE = LANES + 1
UNROLL = 4


def _amf_body(ap_ref, bp_ref, abit_ref, bbit_ref, a_ref, b_ref,
              tab_ref, btab_ref, wcom_ref, out_ref,
              idx_a, idx_b, par_a, par_b, idx_af, idx_bf,
              rows_a, rows_b, ba, bb, wcom_v,
              ps, out_v, sem, sem_bias):
    wid = lax.axis_index("s") * NC + lax.axis_index("c")
    base = wid * BPW

    pltpu.sync_copy(ap_ref.at[pl.ds(base, BPW)], idx_a)
    pltpu.sync_copy(bp_ref.at[pl.ds(base, BPW)], idx_b)
    pltpu.sync_copy(abit_ref.at[pl.ds(base, BPW)], par_a.at[pl.ds(0, BPW)])
    pltpu.sync_copy(bbit_ref.at[pl.ds(base, BPW)], par_b.at[pl.ds(0, BPW)])
    pltpu.sync_copy(a_ref.at[pl.ds(base, BPW)], idx_af)
    pltpu.sync_copy(b_ref.at[pl.ds(base, BPW)], idx_bf)
    pltpu.sync_copy(wcom_ref, wcom_v)

    bias_copies = []
    for k in range(NCHUNK):
        bias_copies.append(pltpu.async_copy(
            btab_ref.at[idx_af.at[pl.ds(k * CHUNK, CHUNK)]],
            ba.at[pl.ds(k * CHUNK, CHUNK)], sem_bias))
        bias_copies.append(pltpu.async_copy(
            btab_ref.at[idx_bf.at[pl.ds(k * CHUNK, CHUNK)]],
            bb.at[pl.ds(k * CHUNK, CHUNK)], sem_bias))

    def fire(k):
        buf = k % 2
        return (pltpu.async_copy(
                    tab_ref.at[idx_a.at[pl.ds(k * CHUNK, CHUNK)]],
                    rows_a.at[buf], sem),
                pltpu.async_copy(
                    tab_ref.at[idx_b.at[pl.ds(k * CHUNK, CHUNK)]],
                    rows_b.at[buf], sem))

    wvs = [wcom_v[pl.ds(c * LANES, LANES)] for c in range(EMB // LANES)]
    lanes = lax.iota(jnp.int32, LANES)
    w0v = wcom_v[pl.ds(EMB, LANES)]
    biasv = wcom_v[pl.ds(EMB + LANES, LANES)]

    pend = fire(0)
    for k in range(NCHUNK):
        for c in pend:
            c.wait()
        if k + 1 < NCHUNK:
            pend = fire(k + 1)
        buf = k % 2

        def elem(i, carry, _k=k, _buf=buf):
            for u in range(UNROLL):
                ii = i * UNROLL + u
                pa = par_a[pl.ds(_k * CHUNK + ii, LANES)][0] * EMB
                pb = par_b[pl.ds(_k * CHUNK + ii, LANES)][0] * EMB
                iv = jnp.full((LANES,), ii, jnp.int32)
                p = None
                for c in range(EMB // LANES):
                    ca = lanes + (pa + c * LANES)
                    cb = lanes + (pb + c * LANES)
                    ra = plsc.load_gather(rows_a.at[_buf], [iv, ca])
                    rb = plsc.load_gather(rows_b.at[_buf], [iv, cb])
                    t = ra * rb * wvs[c]
                    p = t if p is None else p + t
                ps[pl.ds((_k * CHUNK + ii) * PSTRIDE, LANES)] = p
            return carry

        lax.fori_loop(0, CHUNK // UNROLL, elem, 0)

    for c in bias_copies:
        c.wait()

    def group(g, carry):
        ridx = (lanes + g * LANES) * PSTRIDE
        s = None
        for c in range(LANES):
            col = plsc.load_gather(ps, [ridx + c])
            s = col if s is None else s + col
        bav = ba[pl.ds(g * LANES, LANES)]
        bbv = bb[pl.ds(g * LANES, LANES)]
        acc = (bav + bbv) * w0v + biasv + s
        out_v[pl.ds(g * LANES, LANES)] = 1.0 / (1.0 + jnp.exp(-acc))
        return carry

    lax.fori_loop(0, NGROUP, group, 0)
    pltpu.sync_copy(out_v, out_ref.at[pl.ds(base, BPW)])


@functools.partial(jax.jit, static_argnames=())
def kernel(a, b, emb_table, emb_b_table, W, bias):
    ai = a.astype(jnp.int32)
    bi = b.astype(jnp.int32)
    ap = ai >> 1
    bp = bi >> 1
    abit = ai & 1
    bbit = bi & 1
    tab = emb_table.reshape(emb_table.shape[0] // 2, ROWW)
    wcom = jnp.concatenate([
        W[1:, 0],
        jnp.full((LANES,), W[0, 0], jnp.float32),
        jnp.full((LANES,), bias[0], jnp.float32),
    ])

    mesh = plsc.VectorSubcoreMesh(core_axis_name="c", subcore_axis_name="s")
    run = pl.kernel(
        _amf_body,
        out_type=jax.ShapeDtypeStruct((BATCH,), jnp.float32),
        mesh=mesh,
        scratch_types=[
            pltpu.VMEM((BPW,), jnp.int32),
            pltpu.VMEM((BPW,), jnp.int32),
            pltpu.VMEM((BPW + LANES,), jnp.int32),
            pltpu.VMEM((BPW + LANES,), jnp.int32),
            pltpu.VMEM((BPW,), jnp.int32),
            pltpu.VMEM((BPW,), jnp.int32),
            pltpu.VMEM((2, CHUNK, ROWW), jnp.float32),
            pltpu.VMEM((2, CHUNK, ROWW), jnp.float32),
            pltpu.VMEM((BPW,), jnp.float32),
            pltpu.VMEM((BPW,), jnp.float32),
            pltpu.VMEM((EMB + 2 * LANES,), jnp.float32),
            pltpu.VMEM((BPW * PSTRIDE,), jnp.float32),
            pltpu.VMEM((BPW,), jnp.float32),
            pltpu.SemaphoreType.DMA,
            pltpu.SemaphoreType.DMA,
        ],
        compiler_params=pltpu.CompilerParams(
            needs_layout_passes=False, use_tc_tiling_on_sc=True),
    )
    out = run(ap, bp, abit, bbit, ai, bi, tab, emb_b_table.reshape(-1), wcom)
    return out.reshape(BATCH, 1)


# speedup vs baseline: 1.0162x; 1.0162x over previous
import functools

import jax
import jax.numpy as jnp
from jax import lax
from jax.experimental import pallas as pl
from jax.experimental.pallas import tpu as pltpu
from jax.experimental.pallas import tpu_sc as plsc

EMB = 64
BATCH = 16384
NC = 2
NS = 16
NW = NC * NS
BPW = BATCH // NW
CHUNK = 128
NCHUNK = BPW // CHUNK
NGROUP = BPW // 16
LANES = 16
PSTRIDE = LANES + 1


def _amf_body(a_ref, b_ref, tab_ref, btab_ref, wv_ref, w0b_ref, out_ref,
              idx_a, idx_b, rows_a, rows_b, ba, bb, wv_v, w0b_v, ps, out_v,
              sem):
    wid = lax.axis_index("s") * NC + lax.axis_index("c")
    base = wid * BPW

    pltpu.sync_copy(a_ref.at[pl.ds(wid * NCHUNK, NCHUNK)], idx_a)
    pltpu.sync_copy(b_ref.at[pl.ds(wid * NCHUNK, NCHUNK)], idx_b)
    pltpu.sync_copy(wv_ref, wv_v)
    pltpu.sync_copy(w0b_ref, w0b_v)

    copies = []
    for k in range(NCHUNK):
        copies.append(pltpu.async_copy(
            tab_ref.at[idx_a.at[k]], rows_a.at[pl.ds(k * CHUNK, CHUNK)], sem))
        copies.append(pltpu.async_copy(
            tab_ref.at[idx_b.at[k]], rows_b.at[pl.ds(k * CHUNK, CHUNK)], sem))
        copies.append(pltpu.async_copy(
            btab_ref.at[idx_a.at[k]], ba.at[pl.ds(k * CHUNK, CHUNK)], sem))
        copies.append(pltpu.async_copy(
            btab_ref.at[idx_b.at[k]], bb.at[pl.ds(k * CHUNK, CHUNK)], sem))
    for c in copies:
        c.wait()

    wvs = [wv_v[pl.ds(c * LANES, LANES)] for c in range(EMB // LANES)]
    lanes = lax.iota(jnp.int32, LANES)
    w0v = w0b_v[0]
    biasv = w0b_v[1]

    UNROLL = 4

    def elem(i, carry):
        for u in range(UNROLL):
            ii = i * UNROLL + u
            p = None
            for c in range(EMB // LANES):
                ra = rows_a[ii, pl.ds(c * LANES, LANES)]
                rb = rows_b[ii, pl.ds(c * LANES, LANES)]
                t = ra * rb * wvs[c]
                p = t if p is None else p + t
            ps[ii, pl.ds(0, LANES)] = p
        return carry

    lax.fori_loop(0, BPW // UNROLL, elem, 0)

    def group(g, carry):
        ridx = lanes + g * LANES
        s = None
        for c in range(LANES):
            cc = jnp.full((LANES,), c, jnp.int32)
            col = plsc.load_gather(ps, [ridx, cc])
            s = col if s is None else s + col
        bav = ba[pl.ds(g * LANES, LANES)]
        bbv = bb[pl.ds(g * LANES, LANES)]
        acc = (bav + bbv) * w0v + biasv + s
        out_v[pl.ds(g * LANES, LANES)] = 1.0 / (1.0 + jnp.exp(-acc))
        return carry

    lax.fori_loop(0, NGROUP, group, 0)
    pltpu.sync_copy(out_v, out_ref.at[pl.ds(base, BPW)])


@functools.partial(jax.jit, static_argnames=())
def kernel(a, b, emb_table, emb_b_table, W, bias):
    a2 = a.astype(jnp.int32).reshape(NW * NCHUNK, CHUNK)
    b2 = b.astype(jnp.int32).reshape(NW * NCHUNK, CHUNK)
    wv = W[1:, 0]
    w0b = jnp.stack([jnp.full((LANES,), W[0, 0], jnp.float32),
                     jnp.full((LANES,), bias[0], jnp.float32)])

    mesh = plsc.VectorSubcoreMesh(core_axis_name="c", subcore_axis_name="s")
    run = pl.kernel(
        _amf_body,
        out_type=jax.ShapeDtypeStruct((BATCH,), jnp.float32),
        mesh=mesh,
        scratch_types=[
            pltpu.VMEM((NCHUNK, CHUNK), jnp.int32),
            pltpu.VMEM((NCHUNK, CHUNK), jnp.int32),
            pltpu.VMEM((BPW, EMB), jnp.float32),
            pltpu.VMEM((BPW, EMB), jnp.float32),
            pltpu.VMEM((BPW,), jnp.float32),
            pltpu.VMEM((BPW,), jnp.float32),
            pltpu.VMEM((EMB,), jnp.float32),
            pltpu.VMEM((2, LANES), jnp.float32),
            pltpu.VMEM((BPW, PSTRIDE), jnp.float32),
            pltpu.VMEM((BPW,), jnp.float32),
            pltpu.SemaphoreType.DMA,
        ],
        compiler_params=pltpu.CompilerParams(
            needs_layout_passes=False, use_tc_tiling_on_sc=False),
    )
    out = run(a2, b2, emb_table, emb_b_table.reshape(-1), wv, w0b)
    return out.reshape(BATCH, 1)
